# baseline (speedup 1.0000x reference)
import functools

import jax
import jax.numpy as jnp
from jax import lax
from jax.experimental import pallas as pl
from jax.experimental.pallas import tpu as pltpu
from jax.experimental.pallas import tpu_sc as plsc

N = 10000
E = 320000
D = 128
DOUT = 64
G = 64

NSC = 2
NT = 16
NW = NSC * NT
C = 100
J = 100
RA = 10240
RPT = RA // NT

J2 = J // 2
HW = 128

BR = 1000
NG = N // BR


_MESH = plsc.VectorSubcoreMesh(core_axis_name="c", subcore_axis_name="s")


@functools.partial(
    pl.kernel,
    out_type=jax.ShapeDtypeStruct((NSC, RA, HW), jnp.float32),
    mesh=_MESH,
    scratch_types=[
        pltpu.VMEM((J2, C), jnp.int32),
        pltpu.VMEM((C, HW), jnp.float32),
        pltpu.VMEM_SHARED((RA, HW), jnp.float32),
    ],
)
def _sc_hist(dst_hbm, ones_hbm, zrows_hbm, out_hbm, dst_v, ones_v, acc_sp):
    cc = lax.axis_index("c")
    ss = lax.axis_index("s")
    t = cc * NT + ss
    pltpu.sync_copy(zrows_hbm, acc_sp.at[pl.ds(ss * RPT, RPT)])
    pltpu.sync_copy(ones_hbm, ones_v)
    plsc.subcore_barrier()

    for seg in range(2):
        pltpu.sync_copy(dst_hbm.at[t, seg], dst_v)

        def body(j, carry):
            pltpu.sync_copy(ones_v, acc_sp.at[dst_v.at[j]], add=True)
            return carry

        lax.fori_loop(0, J2, body, 0)
    plsc.subcore_barrier()
    pltpu.sync_copy(acc_sp.at[pl.ds(ss * RPT, RPT)],
                    out_hbm.at[cc, pl.ds(ss * RPT, RPT)])


@functools.partial(
    pl.kernel,
    out_type=jax.ShapeDtypeStruct((NSC, RA, D), jnp.float32),
    mesh=_MESH,
    scratch_types=[
        pltpu.VMEM((J2, C), jnp.int32),
        pltpu.VMEM((J2, C), jnp.int32),
        pltpu.VMEM((C, D), jnp.float32),
        pltpu.VMEM((C, D), jnp.float32),
        pltpu.VMEM_SHARED((RA, D), jnp.float32),
        pltpu.SemaphoreType.DMA,
        pltpu.SemaphoreType.DMA,
    ],
)
def _sc_agg(src_hbm, dst_hbm, table_hbm, zrows_hbm, out_hbm,
            src_v, dst_v, rows_a, rows_b, acc_sp, sem_a, sem_b):
    cc = lax.axis_index("c")
    ss = lax.axis_index("s")
    t = cc * NT + ss
    pltpu.sync_copy(zrows_hbm, acc_sp.at[pl.ds(ss * RPT, RPT)])
    plsc.subcore_barrier()

    for seg in range(2):
        pltpu.sync_copy(src_hbm.at[t, seg], src_v)
        pltpu.sync_copy(dst_hbm.at[t, seg], dst_v)
        pltpu.async_copy(table_hbm.at[src_v.at[0]], rows_a, sem_a)

        def body(k, carry):
            ja = 2 * k
            pltpu.async_copy(table_hbm.at[src_v.at[ja + 1]], rows_b, sem_b)
            pltpu.make_async_copy(table_hbm.at[src_v.at[ja]], rows_a,
                                  sem_a).wait()
            pltpu.sync_copy(rows_a, acc_sp.at[dst_v.at[ja]], add=True)
            jn = jnp.minimum(ja + 2, J2 - 1)
            pltpu.async_copy(table_hbm.at[src_v.at[jn]], rows_a, sem_a)
            pltpu.make_async_copy(table_hbm.at[src_v.at[ja + 1]], rows_b,
                                  sem_b).wait()
            pltpu.sync_copy(rows_b, acc_sp.at[dst_v.at[ja + 1]], add=True)
            return carry

        lax.fori_loop(0, J2 // 2, body, 0)
        pltpu.make_async_copy(table_hbm.at[src_v.at[J2 - 1]], rows_a,
                              sem_a).wait()
    plsc.subcore_barrier()
    pltpu.sync_copy(acc_sp.at[pl.ds(ss * RPT, RPT)],
                    out_hbm.at[cc, pl.ds(ss * RPT, RPT)])


def _tc_stage1(hist, x, W1):
    def body(h_ref, x_ref, w_ref, dis_ref, g_ref):
        cnt = jnp.sum(h_ref[0] + h_ref[1], axis=1, keepdims=True) * (1.0 / HW)
        dis = lax.rsqrt(cnt + 1.0)
        h = jnp.dot(x_ref[...], w_ref[...], preferred_element_type=jnp.float32)
        dis_ref[...] = dis
        g_ref[...] = h * dis

    return pl.pallas_call(
        body,
        grid=(NG,),
        in_specs=[
            pl.BlockSpec((NSC, BR, HW), lambda i: (0, i, 0)),
            pl.BlockSpec((BR, D), lambda i: (i, 0)),
            pl.BlockSpec((D, D), lambda i: (0, 0)),
        ],
        out_specs=[
            pl.BlockSpec((BR, 1), lambda i: (i, 0)),
            pl.BlockSpec((BR, D), lambda i: (i, 0)),
        ],
        out_shape=[
            jax.ShapeDtypeStruct((N, 1), jnp.float32),
            jax.ShapeDtypeStruct((N, D), jnp.float32),
        ],
    )(hist, x, W1)


def _tc_stage2(agg, g1, dis, b1, W2):
    def body(a_ref, g_ref, dis_ref, b_ref, w_ref, g2_ref):
        dis = dis_ref[...]
        z = (a_ref[0] + a_ref[1] + g_ref[...]) * dis + b_ref[...]
        z = jnp.maximum(z, 0.0)
        g2_ref[...] = jnp.dot(z, w_ref[...],
                              preferred_element_type=jnp.float32) * dis

    return pl.pallas_call(
        body,
        grid=(NG,),
        in_specs=[
            pl.BlockSpec((NSC, BR, D), lambda i: (0, i, 0)),
            pl.BlockSpec((BR, D), lambda i: (i, 0)),
            pl.BlockSpec((BR, 1), lambda i: (i, 0)),
            pl.BlockSpec((1, D), lambda i: (0, 0)),
            pl.BlockSpec((D, D), lambda i: (0, 0)),
        ],
        out_specs=pl.BlockSpec((BR, D), lambda i: (i, 0)),
        out_shape=jax.ShapeDtypeStruct((N, D), jnp.float32),
    )(agg, g1, dis, b1, W2)


def _tc_stage3(agg, g2, dis, b2, batch_p, lin_W, lin_b):
    def body(a_ref, g_ref, dis_ref, b_ref, bat_ref, w_ref, lb_ref, out_ref,
             sums, cnts):
        i = pl.program_id(0)

        @pl.when(i == 0)
        def _():
            sums[...] = jnp.zeros_like(sums)
            cnts[...] = jnp.zeros_like(cnts)

        z = (a_ref[0] + a_ref[1] + g_ref[...]) * dis_ref[...] + b_ref[...]
        onehot = (bat_ref[...] == lax.broadcasted_iota(
            jnp.int32, (BR, G), 1)).astype(jnp.float32)
        dn = (((0,), (0,)), ((), ()))
        sums[...] += lax.dot_general(onehot, z, dn,
                                     preferred_element_type=jnp.float32)
        cnts[...] += lax.dot_general(onehot, jnp.ones((BR, D), jnp.float32),
                                     dn, preferred_element_type=jnp.float32)

        @pl.when(i == NG - 1)
        def _():
            pooled = sums[...] / jnp.maximum(cnts[...], 1.0)
            out_ref[...] = jnp.dot(pooled, w_ref[...],
                                   preferred_element_type=jnp.float32) + lb_ref[...]

    return pl.pallas_call(
        body,
        grid=(NG,),
        in_specs=[
            pl.BlockSpec((NSC, BR, D), lambda i: (0, i, 0)),
            pl.BlockSpec((BR, D), lambda i: (i, 0)),
            pl.BlockSpec((BR, 1), lambda i: (i, 0)),
            pl.BlockSpec((1, D), lambda i: (0, 0)),
            pl.BlockSpec((BR, 1), lambda i: (i, 0)),
            pl.BlockSpec((D, DOUT), lambda i: (0, 0)),
            pl.BlockSpec((1, DOUT), lambda i: (0, 0)),
        ],
        out_specs=pl.BlockSpec((G, DOUT), lambda i: (0, 0)),
        out_shape=jax.ShapeDtypeStruct((G, DOUT), jnp.float32),
        scratch_shapes=[
            pltpu.VMEM((G, D), jnp.float32),
            pltpu.VMEM((G, D), jnp.float32),
        ],
    )(agg, g2, dis, b2, batch_p, lin_W, lin_b)


def kernel(x, edge_index, batch, W1, b1, W2, b2, lin_W, lin_b):
    ei = edge_index.astype(jnp.int32)
    src_p = ei[0].reshape(NW, 2, J2, C)
    dst_p = ei[1].reshape(NW, 2, J2, C)

    batch_p = batch.astype(jnp.int32).reshape(N, 1)
    ones_c = jnp.ones((C, HW), jnp.float32)
    zrows = jnp.zeros((RPT, D), jnp.float32)

    hist = _sc_hist(dst_p, ones_c, zrows)
    dis, g1 = _tc_stage1(hist, x, W1)
    agg1 = _sc_agg(src_p, dst_p, g1, zrows)
    g2 = _tc_stage2(agg1, g1, dis, b1.reshape(1, D), W2)
    agg2 = _sc_agg(src_p, dst_p, g2, zrows)
    out = _tc_stage3(agg2, g2, dis, b2.reshape(1, D), batch_p,
                     lin_W, lin_b.reshape(1, DOUT))
    return out


# speedup vs baseline: 1.0173x; 1.0173x over previous
import functools

import jax
import jax.numpy as jnp
from jax import lax
from jax.experimental import pallas as pl
from jax.experimental.pallas import tpu as pltpu
from jax.experimental.pallas import tpu_sc as plsc

N = 10000
E = 320000
D = 128
DOUT = 64
G = 64

NSC = 2
NT = 16
NW = NSC * NT
C = 125
J = 80
RA = 10240
RPT = RA // NT

J2 = J // 2
HW = 128

BR = 1000
NG = N // BR


_MESH = plsc.VectorSubcoreMesh(core_axis_name="c", subcore_axis_name="s")


@functools.partial(
    pl.kernel,
    out_type=jax.ShapeDtypeStruct((NSC, RA, HW), jnp.float32),
    mesh=_MESH,
    scratch_types=[
        pltpu.VMEM((J2, C), jnp.int32),
        pltpu.VMEM((C, HW), jnp.float32),
        pltpu.VMEM_SHARED((RA, HW), jnp.float32),
    ],
)
def _sc_hist(dst_hbm, ones_hbm, zrows_hbm, out_hbm, dst_v, ones_v, acc_sp):
    cc = lax.axis_index("c")
    ss = lax.axis_index("s")
    t = cc * NT + ss
    pltpu.sync_copy(zrows_hbm, acc_sp.at[pl.ds(ss * RPT, RPT)])
    pltpu.sync_copy(ones_hbm, ones_v)
    plsc.subcore_barrier()

    for seg in range(2):
        pltpu.sync_copy(dst_hbm.at[t, seg], dst_v)

        def body(j, carry):
            pltpu.sync_copy(ones_v, acc_sp.at[dst_v.at[j]], add=True)
            return carry

        lax.fori_loop(0, J2, body, 0)
    plsc.subcore_barrier()
    pltpu.sync_copy(acc_sp.at[pl.ds(ss * RPT, RPT)],
                    out_hbm.at[cc, pl.ds(ss * RPT, RPT)])


@functools.partial(
    pl.kernel,
    out_type=jax.ShapeDtypeStruct((NSC, RA, D), jnp.float32),
    mesh=_MESH,
    scratch_types=[
        pltpu.VMEM((J2, C), jnp.int32),
        pltpu.VMEM((J2, C), jnp.int32),
        pltpu.VMEM((C, D), jnp.float32),
        pltpu.VMEM((C, D), jnp.float32),
        pltpu.VMEM_SHARED((RA, D), jnp.float32),
        pltpu.SemaphoreType.DMA,
        pltpu.SemaphoreType.DMA,
    ],
)
def _sc_agg(src_hbm, dst_hbm, table_hbm, zrows_hbm, out_hbm,
            src_v, dst_v, rows_a, rows_b, acc_sp, sem_a, sem_b):
    cc = lax.axis_index("c")
    ss = lax.axis_index("s")
    t = cc * NT + ss
    pltpu.sync_copy(zrows_hbm, acc_sp.at[pl.ds(ss * RPT, RPT)])
    plsc.subcore_barrier()

    for seg in range(2):
        pltpu.sync_copy(src_hbm.at[t, seg], src_v)
        pltpu.sync_copy(dst_hbm.at[t, seg], dst_v)
        pltpu.async_copy(table_hbm.at[src_v.at[0]], rows_a, sem_a)

        def body(k, carry):
            ja = 2 * k
            pltpu.async_copy(table_hbm.at[src_v.at[ja + 1]], rows_b, sem_b)
            pltpu.make_async_copy(table_hbm.at[src_v.at[ja]], rows_a,
                                  sem_a).wait()
            pltpu.sync_copy(rows_a, acc_sp.at[dst_v.at[ja]], add=True)
            jn = jnp.minimum(ja + 2, J2 - 1)
            pltpu.async_copy(table_hbm.at[src_v.at[jn]], rows_a, sem_a)
            pltpu.make_async_copy(table_hbm.at[src_v.at[ja + 1]], rows_b,
                                  sem_b).wait()
            pltpu.sync_copy(rows_b, acc_sp.at[dst_v.at[ja + 1]], add=True)
            return carry

        lax.fori_loop(0, J2 // 2, body, 0)
        pltpu.make_async_copy(table_hbm.at[src_v.at[J2 - 1]], rows_a,
                              sem_a).wait()
    plsc.subcore_barrier()
    pltpu.sync_copy(acc_sp.at[pl.ds(ss * RPT, RPT)],
                    out_hbm.at[cc, pl.ds(ss * RPT, RPT)])


def _tc_stage1(hist, x, W1):
    def body(h_ref, x_ref, w_ref, dis_ref, g_ref):
        cnt = jnp.sum(h_ref[0] + h_ref[1], axis=1, keepdims=True) * (1.0 / HW)
        dis = lax.rsqrt(cnt + 1.0)
        h = jnp.dot(x_ref[...], w_ref[...], preferred_element_type=jnp.float32)
        dis_ref[...] = dis
        g_ref[...] = h * dis

    return pl.pallas_call(
        body,
        grid=(NG,),
        in_specs=[
            pl.BlockSpec((NSC, BR, HW), lambda i: (0, i, 0)),
            pl.BlockSpec((BR, D), lambda i: (i, 0)),
            pl.BlockSpec((D, D), lambda i: (0, 0)),
        ],
        out_specs=[
            pl.BlockSpec((BR, 1), lambda i: (i, 0)),
            pl.BlockSpec((BR, D), lambda i: (i, 0)),
        ],
        out_shape=[
            jax.ShapeDtypeStruct((N, 1), jnp.float32),
            jax.ShapeDtypeStruct((N, D), jnp.float32),
        ],
    )(hist, x, W1)


def _tc_stage2(agg, g1, dis, b1, W2):
    def body(a_ref, g_ref, dis_ref, b_ref, w_ref, g2_ref):
        dis = dis_ref[...]
        z = (a_ref[0] + a_ref[1] + g_ref[...]) * dis + b_ref[...]
        z = jnp.maximum(z, 0.0)
        g2_ref[...] = jnp.dot(z, w_ref[...],
                              preferred_element_type=jnp.float32) * dis

    return pl.pallas_call(
        body,
        grid=(NG,),
        in_specs=[
            pl.BlockSpec((NSC, BR, D), lambda i: (0, i, 0)),
            pl.BlockSpec((BR, D), lambda i: (i, 0)),
            pl.BlockSpec((BR, 1), lambda i: (i, 0)),
            pl.BlockSpec((1, D), lambda i: (0, 0)),
            pl.BlockSpec((D, D), lambda i: (0, 0)),
        ],
        out_specs=pl.BlockSpec((BR, D), lambda i: (i, 0)),
        out_shape=jax.ShapeDtypeStruct((N, D), jnp.float32),
    )(agg, g1, dis, b1, W2)


def _tc_stage3(agg, g2, dis, b2, batch_p, lin_W, lin_b):
    def body(a_ref, g_ref, dis_ref, b_ref, bat_ref, w_ref, lb_ref, out_ref,
             sums, cnts):
        i = pl.program_id(0)

        @pl.when(i == 0)
        def _():
            sums[...] = jnp.zeros_like(sums)
            cnts[...] = jnp.zeros_like(cnts)

        z = (a_ref[0] + a_ref[1] + g_ref[...]) * dis_ref[...] + b_ref[...]
        onehot = (bat_ref[...] == lax.broadcasted_iota(
            jnp.int32, (BR, G), 1)).astype(jnp.float32)
        dn = (((0,), (0,)), ((), ()))
        sums[...] += lax.dot_general(onehot, z, dn,
                                     preferred_element_type=jnp.float32)
        cnts[...] += lax.dot_general(onehot, jnp.ones((BR, D), jnp.float32),
                                     dn, preferred_element_type=jnp.float32)

        @pl.when(i == NG - 1)
        def _():
            pooled = sums[...] / jnp.maximum(cnts[...], 1.0)
            out_ref[...] = jnp.dot(pooled, w_ref[...],
                                   preferred_element_type=jnp.float32) + lb_ref[...]

    return pl.pallas_call(
        body,
        grid=(NG,),
        in_specs=[
            pl.BlockSpec((NSC, BR, D), lambda i: (0, i, 0)),
            pl.BlockSpec((BR, D), lambda i: (i, 0)),
            pl.BlockSpec((BR, 1), lambda i: (i, 0)),
            pl.BlockSpec((1, D), lambda i: (0, 0)),
            pl.BlockSpec((BR, 1), lambda i: (i, 0)),
            pl.BlockSpec((D, DOUT), lambda i: (0, 0)),
            pl.BlockSpec((1, DOUT), lambda i: (0, 0)),
        ],
        out_specs=pl.BlockSpec((G, DOUT), lambda i: (0, 0)),
        out_shape=jax.ShapeDtypeStruct((G, DOUT), jnp.float32),
        scratch_shapes=[
            pltpu.VMEM((G, D), jnp.float32),
            pltpu.VMEM((G, D), jnp.float32),
        ],
    )(agg, g2, dis, b2, batch_p, lin_W, lin_b)


def kernel(x, edge_index, batch, W1, b1, W2, b2, lin_W, lin_b):
    ei = edge_index.astype(jnp.int32)
    src_p = ei[0].reshape(NW, 2, J2, C)
    dst_p = ei[1].reshape(NW, 2, J2, C)

    batch_p = batch.astype(jnp.int32).reshape(N, 1)
    ones_c = jnp.ones((C, HW), jnp.float32)
    zrows = jnp.zeros((RPT, D), jnp.float32)

    hist = _sc_hist(dst_p, ones_c, zrows)
    dis, g1 = _tc_stage1(hist, x, W1)
    agg1 = _sc_agg(src_p, dst_p, g1, zrows)
    g2 = _tc_stage2(agg1, g1, dis, b1.reshape(1, D), W2)
    agg2 = _sc_agg(src_p, dst_p, g2, zrows)
    out = _tc_stage3(agg2, g2, dis, b2.reshape(1, D), batch_p,
                     lin_W, lin_b.reshape(1, DOUT))
    return out


# speedup vs baseline: 1.0183x; 1.0010x over previous
import functools

import jax
import jax.numpy as jnp
from jax import lax
from jax.experimental import pallas as pl
from jax.experimental.pallas import tpu as pltpu
from jax.experimental.pallas import tpu_sc as plsc

N = 10000
E = 320000
D = 128
DOUT = 64
G = 64

NSC = 2
NT = 16
NW = NSC * NT
C = 125
J = 80
RA = 10240
RPT = RA // NT

J2 = J // 2
HW = 128

BR = 1000
NG = N // BR


_MESH = plsc.VectorSubcoreMesh(core_axis_name="c", subcore_axis_name="s")


@functools.partial(
    pl.kernel,
    out_type=jax.ShapeDtypeStruct((NSC, RA, HW), jnp.float32),
    mesh=_MESH,
    scratch_types=[
        pltpu.VMEM((J2, C), jnp.int32),
        pltpu.VMEM((C, HW), jnp.float32),
        pltpu.VMEM_SHARED((RA, HW), jnp.float32),
    ],
)
def _sc_hist(dst_hbm, ones_hbm, zrows_hbm, out_hbm, dst_v, ones_v, acc_sp):
    cc = lax.axis_index("c")
    ss = lax.axis_index("s")
    t = cc * NT + ss
    pltpu.sync_copy(zrows_hbm, acc_sp.at[pl.ds(ss * RPT, RPT)])
    pltpu.sync_copy(ones_hbm, ones_v)
    plsc.subcore_barrier()

    for seg in range(2):
        pltpu.sync_copy(dst_hbm.at[t, seg], dst_v)

        def body(j, carry):
            pltpu.sync_copy(ones_v, acc_sp.at[dst_v.at[j]], add=True)
            return carry

        lax.fori_loop(0, J2, body, 0)
    plsc.subcore_barrier()
    pltpu.sync_copy(acc_sp.at[pl.ds(ss * RPT, RPT)],
                    out_hbm.at[cc, pl.ds(ss * RPT, RPT)])


@functools.partial(
    pl.kernel,
    out_type=jax.ShapeDtypeStruct((NSC, RA, D), jnp.float32),
    mesh=_MESH,
    scratch_types=[
        pltpu.VMEM((J2, C), jnp.int32),
        pltpu.VMEM((J2, C), jnp.int32),
        pltpu.VMEM((C, D), jnp.float32),
        pltpu.VMEM((C, D), jnp.float32),
        pltpu.VMEM_SHARED((RA, D), jnp.float32),
        pltpu.SemaphoreType.DMA,
        pltpu.SemaphoreType.DMA,
    ],
)
def _sc_agg(src_hbm, dst_hbm, table_hbm, zrows_hbm, out_hbm,
            src_v, dst_v, rows_a, rows_b, acc_sp, sem_a, sem_b):
    cc = lax.axis_index("c")
    ss = lax.axis_index("s")
    t = cc * NT + ss
    pltpu.sync_copy(zrows_hbm, acc_sp.at[pl.ds(ss * RPT, RPT)])
    plsc.subcore_barrier()

    for seg in range(2):
        pltpu.sync_copy(src_hbm.at[t, seg], src_v)
        pltpu.sync_copy(dst_hbm.at[t, seg], dst_v)
        pltpu.async_copy(table_hbm.at[src_v.at[0]], rows_a, sem_a)

        def body(k, carry):
            ja = 2 * k
            pltpu.async_copy(table_hbm.at[src_v.at[ja + 1]], rows_b, sem_b)
            pltpu.make_async_copy(table_hbm.at[src_v.at[ja]], rows_a,
                                  sem_a).wait()
            pltpu.sync_copy(rows_a, acc_sp.at[dst_v.at[ja]], add=True)
            jn = jnp.minimum(ja + 2, J2 - 1)
            pltpu.async_copy(table_hbm.at[src_v.at[jn]], rows_a, sem_a)
            pltpu.make_async_copy(table_hbm.at[src_v.at[ja + 1]], rows_b,
                                  sem_b).wait()
            pltpu.sync_copy(rows_b, acc_sp.at[dst_v.at[ja + 1]], add=True)
            return carry

        lax.fori_loop(0, J2 // 2, body, 0)
        pltpu.make_async_copy(table_hbm.at[src_v.at[J2 - 1]], rows_a,
                              sem_a).wait()
    plsc.subcore_barrier()
    pltpu.sync_copy(acc_sp.at[pl.ds(ss * RPT, RPT)],
                    out_hbm.at[cc, pl.ds(ss * RPT, RPT)])


def _tc_matmul(x, W1):
    def body(x_ref, w_ref, h_ref):
        h_ref[...] = jnp.dot(x_ref[...], w_ref[...],
                             preferred_element_type=jnp.float32)

    return pl.pallas_call(
        body,
        grid=(NG,),
        in_specs=[
            pl.BlockSpec((BR, D), lambda i: (i, 0)),
            pl.BlockSpec((D, D), lambda i: (0, 0)),
        ],
        out_specs=pl.BlockSpec((BR, D), lambda i: (i, 0)),
        out_shape=jax.ShapeDtypeStruct((N, D), jnp.float32),
    )(x, W1)


def _tc_stage1(hist, h1):
    def body(h_ref, x_ref, dis_ref, g_ref):
        cnt = jnp.sum(h_ref[0] + h_ref[1], axis=1, keepdims=True) * (1.0 / HW)
        dis = lax.rsqrt(cnt + 1.0)
        dis_ref[...] = dis
        g_ref[...] = x_ref[...] * dis

    return pl.pallas_call(
        body,
        grid=(NG,),
        in_specs=[
            pl.BlockSpec((NSC, BR, HW), lambda i: (0, i, 0)),
            pl.BlockSpec((BR, D), lambda i: (i, 0)),
        ],
        out_specs=[
            pl.BlockSpec((BR, 1), lambda i: (i, 0)),
            pl.BlockSpec((BR, D), lambda i: (i, 0)),
        ],
        out_shape=[
            jax.ShapeDtypeStruct((N, 1), jnp.float32),
            jax.ShapeDtypeStruct((N, D), jnp.float32),
        ],
    )(hist, h1)


def _tc_stage2(agg, g1, dis, b1, W2):
    def body(a_ref, g_ref, dis_ref, b_ref, w_ref, g2_ref):
        dis = dis_ref[...]
        z = (a_ref[0] + a_ref[1] + g_ref[...]) * dis + b_ref[...]
        z = jnp.maximum(z, 0.0)
        g2_ref[...] = jnp.dot(z, w_ref[...],
                              preferred_element_type=jnp.float32) * dis

    return pl.pallas_call(
        body,
        grid=(NG,),
        in_specs=[
            pl.BlockSpec((NSC, BR, D), lambda i: (0, i, 0)),
            pl.BlockSpec((BR, D), lambda i: (i, 0)),
            pl.BlockSpec((BR, 1), lambda i: (i, 0)),
            pl.BlockSpec((1, D), lambda i: (0, 0)),
            pl.BlockSpec((D, D), lambda i: (0, 0)),
        ],
        out_specs=pl.BlockSpec((BR, D), lambda i: (i, 0)),
        out_shape=jax.ShapeDtypeStruct((N, D), jnp.float32),
    )(agg, g1, dis, b1, W2)


def _tc_stage3(agg, g2, dis, b2, batch_p, lin_W, lin_b):
    def body(a_ref, g_ref, dis_ref, b_ref, bat_ref, w_ref, lb_ref, out_ref,
             sums, cnts):
        i = pl.program_id(0)

        @pl.when(i == 0)
        def _():
            sums[...] = jnp.zeros_like(sums)
            cnts[...] = jnp.zeros_like(cnts)

        z = (a_ref[0] + a_ref[1] + g_ref[...]) * dis_ref[...] + b_ref[...]
        onehot = (bat_ref[...] == lax.broadcasted_iota(
            jnp.int32, (BR, G), 1)).astype(jnp.float32)
        dn = (((0,), (0,)), ((), ()))
        sums[...] += lax.dot_general(onehot, z, dn,
                                     preferred_element_type=jnp.float32)
        cnts[...] += lax.dot_general(onehot, jnp.ones((BR, D), jnp.float32),
                                     dn, preferred_element_type=jnp.float32)

        @pl.when(i == NG - 1)
        def _():
            pooled = sums[...] / jnp.maximum(cnts[...], 1.0)
            out_ref[...] = jnp.dot(pooled, w_ref[...],
                                   preferred_element_type=jnp.float32) + lb_ref[...]

    return pl.pallas_call(
        body,
        grid=(NG,),
        in_specs=[
            pl.BlockSpec((NSC, BR, D), lambda i: (0, i, 0)),
            pl.BlockSpec((BR, D), lambda i: (i, 0)),
            pl.BlockSpec((BR, 1), lambda i: (i, 0)),
            pl.BlockSpec((1, D), lambda i: (0, 0)),
            pl.BlockSpec((BR, 1), lambda i: (i, 0)),
            pl.BlockSpec((D, DOUT), lambda i: (0, 0)),
            pl.BlockSpec((1, DOUT), lambda i: (0, 0)),
        ],
        out_specs=pl.BlockSpec((G, DOUT), lambda i: (0, 0)),
        out_shape=jax.ShapeDtypeStruct((G, DOUT), jnp.float32),
        scratch_shapes=[
            pltpu.VMEM((G, D), jnp.float32),
            pltpu.VMEM((G, D), jnp.float32),
        ],
    )(agg, g2, dis, b2, batch_p, lin_W, lin_b)


def kernel(x, edge_index, batch, W1, b1, W2, b2, lin_W, lin_b):
    ei = edge_index.astype(jnp.int32)
    src_p = ei[0].reshape(NW, 2, J2, C)
    dst_p = ei[1].reshape(NW, 2, J2, C)

    batch_p = batch.astype(jnp.int32).reshape(N, 1)
    ones_c = jnp.ones((C, HW), jnp.float32)
    zrows = jnp.zeros((RPT, D), jnp.float32)

    hist = _sc_hist(dst_p, ones_c, zrows)
    h1 = _tc_matmul(x, W1)
    dis, g1 = _tc_stage1(hist, h1)
    agg1 = _sc_agg(src_p, dst_p, g1, zrows)
    g2 = _tc_stage2(agg1, g1, dis, b1.reshape(1, D), W2)
    agg2 = _sc_agg(src_p, dst_p, g2, zrows)
    out = _tc_stage3(agg2, g2, dis, b2.reshape(1, D), batch_p,
                     lin_W, lin_b.reshape(1, DOUT))
    return out
